# baseline (speedup 1.0000x reference)
import functools

import jax
import jax.numpy as jnp
from jax import lax
from jax.experimental import pallas as pl
from jax.experimental.pallas import tpu as pltpu
from jax.experimental.pallas import tpu_sc as plsc

N = 10000
DIN = 128
DH = 256
DA = 8
E = 320000
ET = E + N
HALF = 128

NSUB = 16
CH = 64
NB = 324
PT = CH * NB
EP = PT * NSUB
NR = 10240
RPS = NR // NSUB

RB = 1280
GRID = NR // RB

F32 = jnp.float32


def _aug_outputs(h, a_s, a_d, hA_ref, hB_ref, al_ref):
    asv = jnp.sum(h * a_s, axis=1)
    adv = jnp.sum(h * a_d, axis=1)
    hA_ref[...] = h[:, :HALF]
    hB_ref[...] = h[:, HALF:]
    al_ref[...] = jnp.concatenate([asv[:, None], adv[:, None]], axis=1)


def _norm_x(o0_ref, o1_ref, dp_ref):
    den = jnp.sum(dp_ref[...], axis=0)[:, None] + 1e-16
    return jnp.concatenate([o0_ref[...], o1_ref[...]], axis=1) / den


def _stage1_body(obs_ref, w_ref, as_ref, ad_ref, hA_ref, hB_ref, al_ref):
    h = jnp.dot(obs_ref[...], w_ref[...], preferred_element_type=F32)
    _aug_outputs(h, as_ref[...], ad_ref[...], hA_ref, hB_ref, al_ref)


def _stage2_body(o0_ref, o1_ref, dp_ref, b_ref, w_ref, as_ref, ad_ref,
                 hA_ref, hB_ref, al_ref):
    x = jax.nn.relu(_norm_x(o0_ref, o1_ref, dp_ref) + b_ref[...])
    h = jnp.dot(x, w_ref[...], preferred_element_type=F32)
    _aug_outputs(h, as_ref[...], ad_ref[...], hA_ref, hB_ref, al_ref)


def _stage3_body(o0_ref, o1_ref, dp_ref, b2_ref, w3_ref, b3_ref, w4_ref,
                 b4_ref, act_ref):
    x = jax.nn.relu(_norm_x(o0_ref, o1_ref, dp_ref) + b2_ref[...])
    x = jax.nn.relu(jnp.dot(x, w3_ref[...], preferred_element_type=F32)
                    + b3_ref[...])
    act_ref[...] = jnp.tanh(jnp.dot(x, w4_ref[...],
                                    preferred_element_type=F32) + b4_ref[...])


def _row_spec(cols):
    return pl.BlockSpec((RB, cols), lambda i: (i, 0))


def _full_spec(shape):
    return pl.BlockSpec(shape, lambda i: tuple(0 for _ in shape))


_DP_SPEC = pl.BlockSpec((NSUB, RB), lambda i: (0, i))

_AUG_OUT = (
    jax.ShapeDtypeStruct((NR, HALF), F32),
    jax.ShapeDtypeStruct((NR, HALF), F32),
    jax.ShapeDtypeStruct((NR, 2), F32),
)
_AUG_OUT_SPECS = (
    _row_spec(HALF),
    _row_spec(HALF),
    _row_spec(2),
)


def _stage1(obs, w1, a_s, a_d):
    return pl.pallas_call(
        _stage1_body,
        grid=(GRID,),
        in_specs=[_row_spec(DIN), _full_spec((DIN, DH)),
                  _full_spec((1, DH)), _full_spec((1, DH))],
        out_specs=_AUG_OUT_SPECS,
        out_shape=_AUG_OUT,
    )(obs, w1, a_s, a_d)


def _stage2(o0, o1, dp, b, w, a_s, a_d):
    return pl.pallas_call(
        _stage2_body,
        grid=(GRID,),
        in_specs=[_row_spec(HALF), _row_spec(HALF), _DP_SPEC,
                  _full_spec((1, DH)), _full_spec((DH, DH)),
                  _full_spec((1, DH)), _full_spec((1, DH))],
        out_specs=_AUG_OUT_SPECS,
        out_shape=_AUG_OUT,
    )(o0, o1, dp, b, w, a_s, a_d)


def _stage3(o0, o1, dp, b2, w3, b3, w4p, b4p):
    return pl.pallas_call(
        _stage3_body,
        grid=(GRID,),
        in_specs=[_row_spec(HALF), _row_spec(HALF), _DP_SPEC,
                  _full_spec((1, DH)), _full_spec((DH, DH)),
                  _full_spec((1, DH)), _full_spec((DH, HALF)),
                  _full_spec((1, HALF))],
        out_specs=_row_spec(HALF),
        out_shape=jax.ShapeDtypeStruct((NR, HALF), F32),
    )(o0, o1, dp, b2, w3, b3, w4p, b4p)


_MESH = plsc.VectorSubcoreMesh(core_axis_name="c", subcore_axis_name="s")


@functools.partial(
    pl.kernel,
    out_type=(jax.ShapeDtypeStruct((NR, HALF), F32),
              jax.ShapeDtypeStruct((NR, HALF), F32),
              jax.ShapeDtypeStruct((NSUB, NR), F32)),
    mesh=_MESH,
    scratch_types=[
        pltpu.VMEM((2 * NR,), F32),
        pltpu.VMEM((CH,), jnp.int32),
        pltpu.VMEM((CH,), jnp.int32),
        pltpu.VMEM((CH,), jnp.int32),
        pltpu.VMEM((CH,), jnp.int32),
        pltpu.VMEM((CH,), jnp.int32),
        pltpu.VMEM((CH,), jnp.int32),
        pltpu.VMEM((CH, HALF), F32),
        pltpu.VMEM((CH, HALF), F32),
        pltpu.VMEM((NR,), F32),
        pltpu.VMEM_SHARED((NR, HALF), F32),
        pltpu.SemaphoreType.DMA,
        pltpu.SemaphoreType.DMA,
        pltpu.SemaphoreType.DMA,
        pltpu.SemaphoreType.DMA,
        pltpu.SemaphoreType.DMA,
        pltpu.SemaphoreType.DMA,
    ],
    compiler_params=pltpu.CompilerParams(needs_layout_passes=False),
)
def _sc_aggregate(hA, hB, al, src_hbm, dst_hbm, z_hbm, out0, out1, dpart,
                  al_v, src0, src1, dst0, dst1, dstS0, dstS1, rows0, rows1,
                  den_v, acc, sG0, sG1, sS0, sS1, sI0, sI1):
    c = lax.axis_index("c")
    s = lax.axis_index("s")
    base = s * PT

    srcs = (src0, src1)
    dsts = (dst0, dst1)
    dstSs = (dstS0, dstS1)
    rows = (rows0, rows1)
    sG = (sG0, sG1)
    sS = (sS0, sS1)
    sI = (sI0, sI1)

    def gather_issue(q):
        @pl.when(c == 0)
        def _():
            pltpu.async_copy(hA.at[srcs[q]], rows[q], sG[q])

        @pl.when(c == 1)
        def _():
            pltpu.async_copy(hB.at[srcs[q]], rows[q], sG[q])

    def gather_wait(q):
        @pl.when(c == 0)
        def _():
            pltpu.make_async_copy(hA.at[srcs[q]], rows[q], sG[q]).wait()

        @pl.when(c == 1)
        def _():
            pltpu.make_async_copy(hB.at[srcs[q]], rows[q], sG[q]).wait()

    def scatter_issue(q):
        pltpu.async_copy(rows[q], acc.at[dstSs[q]], sS[q], add=True)

    def scatter_wait(q):
        pltpu.make_async_copy(rows[q], acc.at[dstSs[q]], sS[q]).wait()

    def idx_issue(b, q):
        off = base + b * CH
        pltpu.async_copy(src_hbm.at[pl.ds(off, CH)], srcs[q], sI[q])
        pltpu.async_copy(dst_hbm.at[pl.ds(off, CH)], dsts[q], sI[q])

    def idx_wait(b, q):
        off = base + b * CH
        pltpu.make_async_copy(src_hbm.at[pl.ds(off, CH)], srcs[q],
                              sI[q]).wait()
        pltpu.make_async_copy(dst_hbm.at[pl.ds(off, CH)], dsts[q],
                              sI[q]).wait()

    pltpu.sync_copy(al, al_v)
    pltpu.sync_copy(z_hbm.at[pl.ds(s * RPS, RPS)],
                    acc.at[pl.ds(s * RPS, RPS)])
    pltpu.sync_copy(src_hbm.at[pl.ds(base, CH)], src0)
    pltpu.sync_copy(dst_hbm.at[pl.ds(base, CH)], dst0)
    gather_issue(0)
    idx_issue(1, 1)

    zeros16 = jnp.zeros((16,), F32)
    lanes = lax.iota(jnp.int32, 16)

    def zden(v, carry):
        den_v[pl.ds(v * 16, 16)] = zeros16
        return carry

    lax.fori_loop(0, NR // 16, zden, 0)
    plsc.subcore_barrier()

    def block_steps(b, q):
        off = base + b * CH
        gather_wait(q)

        @pl.when(b < NB - 1)
        def _():
            idx_wait(b + 1, 1 - q)

        @pl.when(b >= 1)
        def _():
            scatter_wait(1 - q)

        @pl.when(b < NB - 1)
        def _():
            gather_issue(1 - q)

        for j in range(CH // 16):
            sv = srcs[q][pl.ds(j * 16, 16)]
            dv = dsts[q][pl.ds(j * 16, 16)]
            a = (plsc.load_gather(al_v, [sv * 2])
                 + plsc.load_gather(al_v, [dv * 2 + 1]))
            a = jnp.maximum(a, 0.2 * a)
            e = jnp.exp(a)
            gidx = off + j * 16 + lanes
            e = jnp.where(gidx < ET, e, 0.0)

            for k in range(16):
                ev = e.at[jnp.full((16,), k, jnp.int32)].get(
                    mode="promise_in_bounds")
                r = j * 16 + k
                for v in range(HALF // 16):
                    rows[q][r, pl.ds(v * 16, 16)] = (
                        rows[q][r, pl.ds(v * 16, 16)] * ev)

            dk, ev = plsc.sort_key_val(dv, e)
            psum = plsc.cumsum(ev)
            dnext = dk.at[jnp.minimum(lanes + 1, 15)].get(
                mode="promise_in_bounds")
            mend = (lanes == 15) | (dk != dnext)
            dprev = dk.at[jnp.maximum(lanes - 1, 0)].get(
                mode="promise_in_bounds")
            sbeg = (lanes == 0) | (dk != dprev)
            sidx = plsc.cummax(jnp.where(sbeg, lanes, 0))
            pprev = psum.at[jnp.maximum(sidx - 1, 0)].get(
                mode="promise_in_bounds")
            runsum = psum - jnp.where(sidx == 0, 0.0, pprev)
            plsc.addupdate_scatter(den_v, [dk], runsum, mask=mend)

        def dmove(v, carry3):
            dstSs[q][pl.ds(v * 16, 16)] = dsts[q][pl.ds(v * 16, 16)]
            return carry3

        lax.fori_loop(0, CH // 16, dmove, 0)
        scatter_issue(q)

        @pl.when(b < NB - 2)
        def _():
            idx_issue(b + 2, q)

    def block_body(b, carry):
        @pl.when(b % 2 == 0)
        def _():
            block_steps(b, 0)

        @pl.when(b % 2 == 1)
        def _():
            block_steps(b, 1)

        return carry

    lax.fori_loop(0, NB, block_body, 0)
    scatter_wait((NB - 1) % 2)

    @pl.when(c == 0)
    def _():
        pltpu.sync_copy(den_v, dpart.at[s])

    plsc.subcore_barrier()

    @pl.when(c == 0)
    def _():
        pltpu.sync_copy(acc.at[pl.ds(s * RPS, RPS)],
                        out0.at[pl.ds(s * RPS, RPS)])

    @pl.when(c == 1)
    def _():
        pltpu.sync_copy(acc.at[pl.ds(s * RPS, RPS)],
                        out1.at[pl.ds(s * RPS, RPS)])


def kernel(obs, edge_index, W1, as1, ad1, b1, W2, as2, ad2, b2, W3, b3, W4,
           b4):
    loop = jnp.arange(N, dtype=jnp.int32)
    pad = jnp.zeros((EP - ET,), jnp.int32)
    src = jnp.concatenate([edge_index[0].astype(jnp.int32), loop, pad])
    dst = jnp.concatenate([edge_index[1].astype(jnp.int32), loop, pad])
    z = jnp.zeros((NR, HALF), F32)
    obs_p = jnp.pad(obs, ((0, NR - N), (0, 0)))

    as1r = as1.reshape(1, DH)
    ad1r = ad1.reshape(1, DH)
    as2r = as2.reshape(1, DH)
    ad2r = ad2.reshape(1, DH)
    b1r = b1.reshape(1, DH)
    b2r = b2.reshape(1, DH)
    b3r = b3.reshape(1, DH)
    w4p = jnp.pad(W4, ((0, 0), (0, HALF - DA)))
    b4p = jnp.pad(b4, (0, HALF - DA)).reshape(1, HALF)

    hA, hB, al = _stage1(obs_p, W1, as1r, ad1r)
    o0, o1, d1 = _sc_aggregate(hA, hB, al.reshape(2 * NR), src, dst, z)
    hA2, hB2, al2 = _stage2(o0, o1, d1, b1r, W2, as2r, ad2r)
    p0, p1, d2 = _sc_aggregate(hA2, hB2, al2.reshape(2 * NR), src, dst, z)
    act = _stage3(p0, p1, d2, b2r, W3, b3r, w4p, b4p)
    return act[:N, :DA]


# speedup vs baseline: 1.1311x; 1.1311x over previous
import functools

import jax
import jax.numpy as jnp
from jax import lax
from jax.experimental import pallas as pl
from jax.experimental.pallas import tpu as pltpu
from jax.experimental.pallas import tpu_sc as plsc

N = 10000
DIN = 128
DH = 256
DA = 8
E = 320000
ET = E + N
HALF = 128

NSUB = 16
CH = 32
NB = 648
NPAR = 4
PT = CH * NB
EP = PT * NSUB
NR = 10240
RPS = NR // NSUB

RB = 1280
GRID = NR // RB

F32 = jnp.float32


def _aug_outputs(h, a_s, a_d, hA_ref, hB_ref, al_ref):
    asv = jnp.sum(h * a_s, axis=1)
    adv = jnp.sum(h * a_d, axis=1)
    hA_ref[...] = h[:, :HALF]
    hB_ref[...] = h[:, HALF:]
    al_ref[...] = jnp.concatenate([asv[:, None], adv[:, None]], axis=1)


def _norm_x(o0_ref, o1_ref, dp_ref):
    den = jnp.sum(dp_ref[...], axis=0)[:, None] + 1e-16
    return jnp.concatenate([o0_ref[...], o1_ref[...]], axis=1) / den


def _stage1_body(obs_ref, w_ref, as_ref, ad_ref, hA_ref, hB_ref, al_ref):
    h = jnp.dot(obs_ref[...], w_ref[...], preferred_element_type=F32)
    _aug_outputs(h, as_ref[...], ad_ref[...], hA_ref, hB_ref, al_ref)


def _stage2_body(o0_ref, o1_ref, dp_ref, b_ref, w_ref, as_ref, ad_ref,
                 hA_ref, hB_ref, al_ref):
    x = jax.nn.relu(_norm_x(o0_ref, o1_ref, dp_ref) + b_ref[...])
    h = jnp.dot(x, w_ref[...], preferred_element_type=F32)
    _aug_outputs(h, as_ref[...], ad_ref[...], hA_ref, hB_ref, al_ref)


def _stage3_body(o0_ref, o1_ref, dp_ref, b2_ref, w3_ref, b3_ref, w4_ref,
                 b4_ref, act_ref):
    x = jax.nn.relu(_norm_x(o0_ref, o1_ref, dp_ref) + b2_ref[...])
    x = jax.nn.relu(jnp.dot(x, w3_ref[...], preferred_element_type=F32)
                    + b3_ref[...])
    act_ref[...] = jnp.tanh(jnp.dot(x, w4_ref[...],
                                    preferred_element_type=F32) + b4_ref[...])


def _row_spec(cols):
    return pl.BlockSpec((RB, cols), lambda i: (i, 0))


def _full_spec(shape):
    return pl.BlockSpec(shape, lambda i: tuple(0 for _ in shape))


_DP_SPEC = pl.BlockSpec((NSUB, RB), lambda i: (0, i))

_AUG_OUT = (
    jax.ShapeDtypeStruct((NR, HALF), F32),
    jax.ShapeDtypeStruct((NR, HALF), F32),
    jax.ShapeDtypeStruct((NR, 2), F32),
)
_AUG_OUT_SPECS = (
    _row_spec(HALF),
    _row_spec(HALF),
    _row_spec(2),
)


def _stage1(obs, w1, a_s, a_d):
    return pl.pallas_call(
        _stage1_body,
        grid=(GRID,),
        in_specs=[_row_spec(DIN), _full_spec((DIN, DH)),
                  _full_spec((1, DH)), _full_spec((1, DH))],
        out_specs=_AUG_OUT_SPECS,
        out_shape=_AUG_OUT,
    )(obs, w1, a_s, a_d)


def _stage2(o0, o1, dp, b, w, a_s, a_d):
    return pl.pallas_call(
        _stage2_body,
        grid=(GRID,),
        in_specs=[_row_spec(HALF), _row_spec(HALF), _DP_SPEC,
                  _full_spec((1, DH)), _full_spec((DH, DH)),
                  _full_spec((1, DH)), _full_spec((1, DH))],
        out_specs=_AUG_OUT_SPECS,
        out_shape=_AUG_OUT,
    )(o0, o1, dp, b, w, a_s, a_d)


def _stage3(o0, o1, dp, b2, w3, b3, w4p, b4p):
    return pl.pallas_call(
        _stage3_body,
        grid=(GRID,),
        in_specs=[_row_spec(HALF), _row_spec(HALF), _DP_SPEC,
                  _full_spec((1, DH)), _full_spec((DH, DH)),
                  _full_spec((1, DH)), _full_spec((DH, HALF)),
                  _full_spec((1, HALF))],
        out_specs=_row_spec(HALF),
        out_shape=jax.ShapeDtypeStruct((NR, HALF), F32),
    )(o0, o1, dp, b2, w3, b3, w4p, b4p)


_MESH = plsc.VectorSubcoreMesh(core_axis_name="c", subcore_axis_name="s")


@functools.partial(
    pl.kernel,
    out_type=(jax.ShapeDtypeStruct((NR, HALF), F32),
              jax.ShapeDtypeStruct((NR, HALF), F32),
              jax.ShapeDtypeStruct((NSUB, NR), F32)),
    mesh=_MESH,
    scratch_types=(
        [pltpu.VMEM((2 * NR,), F32)]
        + [pltpu.VMEM((CH,), jnp.int32)] * NPAR
        + [pltpu.VMEM((CH,), jnp.int32)] * NPAR
        + [pltpu.VMEM((CH,), jnp.int32)] * NPAR
        + [pltpu.VMEM((CH, HALF), F32)] * NPAR
        + [pltpu.VMEM((NR,), F32),
           pltpu.VMEM_SHARED((NR, HALF), F32)]
        + [pltpu.SemaphoreType.DMA] * (3 * NPAR)
    ),
    compiler_params=pltpu.CompilerParams(needs_layout_passes=False),
)
def _sc_aggregate(hA, hB, al, src_hbm, dst_hbm, z_hbm, out0, out1, dpart,
                  al_v, src0, src1, src2, src3, dst0, dst1, dst2, dst3,
                  dstS0, dstS1, dstS2, dstS3, rows0, rows1, rows2, rows3,
                  den_v, acc, sG0, sG1, sG2, sG3, sS0, sS1, sS2, sS3,
                  sI0, sI1, sI2, sI3):
    c = lax.axis_index("c")
    s = lax.axis_index("s")
    base = s * PT

    srcs = (src0, src1, src2, src3)
    dsts = (dst0, dst1, dst2, dst3)
    dstSs = (dstS0, dstS1, dstS2, dstS3)
    rows = (rows0, rows1, rows2, rows3)
    sG = (sG0, sG1, sG2, sG3)
    sS = (sS0, sS1, sS2, sS3)
    sI = (sI0, sI1, sI2, sI3)

    def gather_issue(q):
        @pl.when(c == 0)
        def _():
            pltpu.async_copy(hA.at[srcs[q]], rows[q], sG[q])

        @pl.when(c == 1)
        def _():
            pltpu.async_copy(hB.at[srcs[q]], rows[q], sG[q])

    def gather_wait(q):
        @pl.when(c == 0)
        def _():
            pltpu.make_async_copy(hA.at[srcs[q]], rows[q], sG[q]).wait()

        @pl.when(c == 1)
        def _():
            pltpu.make_async_copy(hB.at[srcs[q]], rows[q], sG[q]).wait()

    def scatter_issue(q):
        pltpu.async_copy(rows[q], acc.at[dstSs[q]], sS[q], add=True)

    def scatter_wait(q):
        pltpu.make_async_copy(rows[q], acc.at[dstSs[q]], sS[q]).wait()

    def idx_issue(b, q):
        off = base + b * CH
        pltpu.async_copy(src_hbm.at[pl.ds(off, CH)], srcs[q], sI[q])
        pltpu.async_copy(dst_hbm.at[pl.ds(off, CH)], dsts[q], sI[q])

    def idx_wait(b, q):
        off = base + b * CH
        pltpu.make_async_copy(src_hbm.at[pl.ds(off, CH)], srcs[q],
                              sI[q]).wait()
        pltpu.make_async_copy(dst_hbm.at[pl.ds(off, CH)], dsts[q],
                              sI[q]).wait()

    pltpu.sync_copy(al, al_v)
    pltpu.sync_copy(z_hbm.at[pl.ds(s * RPS, RPS)],
                    acc.at[pl.ds(s * RPS, RPS)])
    pltpu.sync_copy(src_hbm.at[pl.ds(base, CH)], src0)
    pltpu.sync_copy(dst_hbm.at[pl.ds(base, CH)], dst0)
    pltpu.sync_copy(src_hbm.at[pl.ds(base + CH, CH)], src1)
    pltpu.sync_copy(dst_hbm.at[pl.ds(base + CH, CH)], dst1)
    gather_issue(0)
    gather_issue(1)
    idx_issue(2, 2)
    idx_issue(3, 3)

    zeros16 = jnp.zeros((16,), F32)
    lanes = lax.iota(jnp.int32, 16)

    def zden(v, carry):
        den_v[pl.ds(v * 16, 16)] = zeros16
        return carry

    lax.fori_loop(0, NR // 16, zden, 0)
    plsc.subcore_barrier()

    def block_steps(b, q):
        off = base + b * CH
        q2 = (q + 2) % NPAR
        gather_wait(q)

        @pl.when(b >= 2)
        def _():
            scatter_wait(q2)

        @pl.when(b < NB - 2)
        def _():
            idx_wait(b + 2, q2)
            gather_issue(q2)

        for j in range(CH // 16):
            sv = srcs[q][pl.ds(j * 16, 16)]
            dv = dsts[q][pl.ds(j * 16, 16)]
            a = (plsc.load_gather(al_v, [sv * 2])
                 + plsc.load_gather(al_v, [dv * 2 + 1]))
            a = jnp.maximum(a, 0.2 * a)
            e = jnp.exp(a)
            gidx = off + j * 16 + lanes
            e = jnp.where(gidx < ET, e, 0.0)

            for k in range(16):
                ev = e.at[jnp.full((16,), k, jnp.int32)].get(
                    mode="promise_in_bounds")
                r = j * 16 + k
                for v in range(HALF // 16):
                    rows[q][r, pl.ds(v * 16, 16)] = (
                        rows[q][r, pl.ds(v * 16, 16)] * ev)

            dk, ev = plsc.sort_key_val(dv, e)
            psum = plsc.cumsum(ev)
            dnext = dk.at[jnp.minimum(lanes + 1, 15)].get(
                mode="promise_in_bounds")
            mend = (lanes == 15) | (dk != dnext)
            dprev = dk.at[jnp.maximum(lanes - 1, 0)].get(
                mode="promise_in_bounds")
            sbeg = (lanes == 0) | (dk != dprev)
            sidx = plsc.cummax(jnp.where(sbeg, lanes, 0))
            pprev = psum.at[jnp.maximum(sidx - 1, 0)].get(
                mode="promise_in_bounds")
            runsum = psum - jnp.where(sidx == 0, 0.0, pprev)
            plsc.addupdate_scatter(den_v, [dk], runsum, mask=mend)

        def dmove(v, carry3):
            dstSs[q][pl.ds(v * 16, 16)] = dsts[q][pl.ds(v * 16, 16)]
            return carry3

        lax.fori_loop(0, CH // 16, dmove, 0)
        scatter_issue(q)

        @pl.when(b < NB - 4)
        def _():
            idx_issue(b + 4, q)

    def block_body(b, carry):
        for qq in range(NPAR):
            @pl.when(b % NPAR == qq)
            def _(qq=qq):
                block_steps(b, qq)

        return carry

    lax.fori_loop(0, NB, block_body, 0)
    scatter_wait((NB - 2) % NPAR)
    scatter_wait((NB - 1) % NPAR)

    @pl.when(c == 0)
    def _():
        pltpu.sync_copy(den_v, dpart.at[s])

    plsc.subcore_barrier()

    @pl.when(c == 0)
    def _():
        pltpu.sync_copy(acc.at[pl.ds(s * RPS, RPS)],
                        out0.at[pl.ds(s * RPS, RPS)])

    @pl.when(c == 1)
    def _():
        pltpu.sync_copy(acc.at[pl.ds(s * RPS, RPS)],
                        out1.at[pl.ds(s * RPS, RPS)])


def kernel(obs, edge_index, W1, as1, ad1, b1, W2, as2, ad2, b2, W3, b3, W4,
           b4):
    loop = jnp.arange(N, dtype=jnp.int32)
    pad = jnp.zeros((EP - ET,), jnp.int32)
    src = jnp.concatenate([edge_index[0].astype(jnp.int32), loop, pad])
    dst = jnp.concatenate([edge_index[1].astype(jnp.int32), loop, pad])
    z = jnp.zeros((NR, HALF), F32)
    obs_p = jnp.pad(obs, ((0, NR - N), (0, 0)))

    as1r = as1.reshape(1, DH)
    ad1r = ad1.reshape(1, DH)
    as2r = as2.reshape(1, DH)
    ad2r = ad2.reshape(1, DH)
    b1r = b1.reshape(1, DH)
    b2r = b2.reshape(1, DH)
    b3r = b3.reshape(1, DH)
    w4p = jnp.pad(W4, ((0, 0), (0, HALF - DA)))
    b4p = jnp.pad(b4, (0, HALF - DA)).reshape(1, HALF)

    hA, hB, al = _stage1(obs_p, W1, as1r, ad1r)
    o0, o1, d1 = _sc_aggregate(hA, hB, al.reshape(2 * NR), src, dst, z)
    hA2, hB2, al2 = _stage2(o0, o1, d1, b1r, W2, as2r, ad2r)
    p0, p1, d2 = _sc_aggregate(hA2, hB2, al2.reshape(2 * NR), src, dst, z)
    act = _stage3(p0, p1, d2, b2r, W3, b3r, w4p, b4p)
    return act[:N, :DA]


# speedup vs baseline: 1.1332x; 1.0018x over previous
import functools

import jax
import jax.numpy as jnp
from jax import lax
from jax.experimental import pallas as pl
from jax.experimental.pallas import tpu as pltpu
from jax.experimental.pallas import tpu_sc as plsc

N = 10000
DIN = 128
DH = 256
DA = 8
E = 320000
ET = E + N
HALF = 128

NSUB = 16
CH = 32
NB = 648
NPAR = 4
PT = CH * NB
EP = PT * NSUB
NR = 10240
RPS = NR // NSUB

RB = 1280
GRID = NR // RB

F32 = jnp.float32


def _aug_outputs(h, a_s, a_d, hA_ref, hB_ref, al_ref):
    asv = jnp.sum(h * a_s, axis=1)
    adv = jnp.sum(h * a_d, axis=1)
    hA_ref[...] = h[:, :HALF]
    hB_ref[...] = h[:, HALF:]
    al_ref[...] = jnp.concatenate([asv[:, None], adv[:, None]], axis=1)


def _norm_x(o0_ref, o1_ref, dp_ref):
    den = jnp.sum(dp_ref[...], axis=0)[:, None] + 1e-16
    return jnp.concatenate([o0_ref[...], o1_ref[...]], axis=1) / den


def _stage1_body(obs_ref, w_ref, as_ref, ad_ref, hA_ref, hB_ref, al_ref):
    h = jnp.dot(obs_ref[...], w_ref[...], preferred_element_type=F32)
    _aug_outputs(h, as_ref[...], ad_ref[...], hA_ref, hB_ref, al_ref)


def _stage2_body(o0_ref, o1_ref, dp_ref, b_ref, w_ref, as_ref, ad_ref,
                 hA_ref, hB_ref, al_ref):
    x = jax.nn.relu(_norm_x(o0_ref, o1_ref, dp_ref) + b_ref[...])
    h = jnp.dot(x, w_ref[...], preferred_element_type=F32)
    _aug_outputs(h, as_ref[...], ad_ref[...], hA_ref, hB_ref, al_ref)


def _stage3_body(o0_ref, o1_ref, dp_ref, b2_ref, w3_ref, b3_ref, w4_ref,
                 b4_ref, act_ref):
    x = jax.nn.relu(_norm_x(o0_ref, o1_ref, dp_ref) + b2_ref[...])
    x = jax.nn.relu(jnp.dot(x, w3_ref[...], preferred_element_type=F32)
                    + b3_ref[...])
    act_ref[...] = jnp.tanh(jnp.dot(x, w4_ref[...],
                                    preferred_element_type=F32) + b4_ref[...])


def _row_spec(cols):
    return pl.BlockSpec((RB, cols), lambda i: (i, 0))


def _full_spec(shape):
    return pl.BlockSpec(shape, lambda i: tuple(0 for _ in shape))


_DP_SPEC = pl.BlockSpec((NSUB, RB), lambda i: (0, i))

_AUG_OUT = (
    jax.ShapeDtypeStruct((NR, HALF), F32),
    jax.ShapeDtypeStruct((NR, HALF), F32),
    jax.ShapeDtypeStruct((NR, 2), F32),
)
_AUG_OUT_SPECS = (
    _row_spec(HALF),
    _row_spec(HALF),
    _row_spec(2),
)


def _stage1(obs, w1, a_s, a_d):
    return pl.pallas_call(
        _stage1_body,
        grid=(GRID,),
        in_specs=[_row_spec(DIN), _full_spec((DIN, DH)),
                  _full_spec((1, DH)), _full_spec((1, DH))],
        out_specs=_AUG_OUT_SPECS,
        out_shape=_AUG_OUT,
    )(obs, w1, a_s, a_d)


def _stage2(o0, o1, dp, b, w, a_s, a_d):
    return pl.pallas_call(
        _stage2_body,
        grid=(GRID,),
        in_specs=[_row_spec(HALF), _row_spec(HALF), _DP_SPEC,
                  _full_spec((1, DH)), _full_spec((DH, DH)),
                  _full_spec((1, DH)), _full_spec((1, DH))],
        out_specs=_AUG_OUT_SPECS,
        out_shape=_AUG_OUT,
    )(o0, o1, dp, b, w, a_s, a_d)


def _stage3(o0, o1, dp, b2, w3, b3, w4p, b4p):
    return pl.pallas_call(
        _stage3_body,
        grid=(GRID,),
        in_specs=[_row_spec(HALF), _row_spec(HALF), _DP_SPEC,
                  _full_spec((1, DH)), _full_spec((DH, DH)),
                  _full_spec((1, DH)), _full_spec((DH, HALF)),
                  _full_spec((1, HALF))],
        out_specs=_row_spec(HALF),
        out_shape=jax.ShapeDtypeStruct((NR, HALF), F32),
    )(o0, o1, dp, b2, w3, b3, w4p, b4p)


_MESH = plsc.VectorSubcoreMesh(core_axis_name="c", subcore_axis_name="s")


@functools.partial(
    pl.kernel,
    out_type=(jax.ShapeDtypeStruct((NR, HALF), F32),
              jax.ShapeDtypeStruct((NR, HALF), F32),
              jax.ShapeDtypeStruct((NSUB, NR), F32)),
    mesh=_MESH,
    scratch_types=(
        [pltpu.VMEM((2 * NR,), F32)]
        + [pltpu.VMEM((CH,), jnp.int32)] * NPAR
        + [pltpu.VMEM((CH,), jnp.int32)] * NPAR
        + [pltpu.VMEM((CH,), jnp.int32)] * NPAR
        + [pltpu.VMEM((CH, HALF), F32)] * NPAR
        + [pltpu.VMEM((NR,), F32),
           pltpu.VMEM_SHARED((NR, HALF), F32)]
        + [pltpu.SemaphoreType.DMA] * (3 * NPAR)
    ),
    compiler_params=pltpu.CompilerParams(needs_layout_passes=False),
)
def _sc_aggregate(hA, hB, al, src_hbm, dst_hbm, z_hbm, out0, out1, dpart,
                  al_v, src0, src1, src2, src3, dst0, dst1, dst2, dst3,
                  dstS0, dstS1, dstS2, dstS3, rows0, rows1, rows2, rows3,
                  den_v, acc, sG0, sG1, sG2, sG3, sS0, sS1, sS2, sS3,
                  sI0, sI1, sI2, sI3):
    c = lax.axis_index("c")
    s = lax.axis_index("s")
    base = s * PT

    srcs = (src0, src1, src2, src3)
    dsts = (dst0, dst1, dst2, dst3)
    dstSs = (dstS0, dstS1, dstS2, dstS3)
    rows = (rows0, rows1, rows2, rows3)
    sG = (sG0, sG1, sG2, sG3)
    sS = (sS0, sS1, sS2, sS3)
    sI = (sI0, sI1, sI2, sI3)

    def gather_issue(q):
        @pl.when(c == 0)
        def _():
            pltpu.async_copy(hA.at[srcs[q]], rows[q], sG[q])

        @pl.when(c == 1)
        def _():
            pltpu.async_copy(hB.at[srcs[q]], rows[q], sG[q])

    def gather_wait(q):
        @pl.when(c == 0)
        def _():
            pltpu.make_async_copy(hA.at[srcs[q]], rows[q], sG[q]).wait()

        @pl.when(c == 1)
        def _():
            pltpu.make_async_copy(hB.at[srcs[q]], rows[q], sG[q]).wait()

    def scatter_issue(q):
        pltpu.async_copy(rows[q], acc.at[dstSs[q]], sS[q], add=True)

    def scatter_wait(q):
        pltpu.make_async_copy(rows[q], acc.at[dstSs[q]], sS[q]).wait()

    def idx_issue(b, q):
        off = base + b * CH
        pltpu.async_copy(src_hbm.at[pl.ds(off, CH)], srcs[q], sI[q])
        pltpu.async_copy(dst_hbm.at[pl.ds(off, CH)], dsts[q], sI[q])

    def idx_wait(b, q):
        off = base + b * CH
        pltpu.make_async_copy(src_hbm.at[pl.ds(off, CH)], srcs[q],
                              sI[q]).wait()
        pltpu.make_async_copy(dst_hbm.at[pl.ds(off, CH)], dsts[q],
                              sI[q]).wait()

    pltpu.sync_copy(al, al_v)
    pltpu.sync_copy(z_hbm.at[pl.ds(s * RPS, RPS)],
                    acc.at[pl.ds(s * RPS, RPS)])
    pltpu.sync_copy(src_hbm.at[pl.ds(base, CH)], src0)
    pltpu.sync_copy(dst_hbm.at[pl.ds(base, CH)], dst0)
    pltpu.sync_copy(src_hbm.at[pl.ds(base + CH, CH)], src1)
    pltpu.sync_copy(dst_hbm.at[pl.ds(base + CH, CH)], dst1)
    gather_issue(0)
    gather_issue(1)
    idx_issue(2, 2)
    idx_issue(3, 3)

    zeros16 = jnp.zeros((16,), F32)
    lanes = lax.iota(jnp.int32, 16)

    def zden(v, carry):
        den_v[pl.ds(v * 16, 16)] = zeros16
        return carry

    lax.fori_loop(0, NR // 16, zden, 0)
    plsc.subcore_barrier()

    def block_steps(b, q):
        off = base + b * CH
        q2 = (q + 2) % NPAR
        gather_wait(q)

        @pl.when(b < NB - 2)
        def _():
            idx_wait(b + 2, q2)
            gather_issue(q2)

        for j in range(CH // 16):
            sv = srcs[q][pl.ds(j * 16, 16)]
            dv = dsts[q][pl.ds(j * 16, 16)]
            a = (plsc.load_gather(al_v, [sv * 2])
                 + plsc.load_gather(al_v, [dv * 2 + 1]))
            a = jnp.maximum(a, 0.2 * a)
            e = jnp.exp(a)
            gidx = off + j * 16 + lanes
            e = jnp.where(gidx < ET, e, 0.0)

            for k in range(16):
                ev = e.at[jnp.full((16,), k, jnp.int32)].get(
                    mode="promise_in_bounds")
                r = j * 16 + k
                for v in range(HALF // 16):
                    rows[q][r, pl.ds(v * 16, 16)] = (
                        rows[q][r, pl.ds(v * 16, 16)] * ev)

            dk, ev = plsc.sort_key_val(dv, e)
            psum = plsc.cumsum(ev)
            dnext = dk.at[jnp.minimum(lanes + 1, 15)].get(
                mode="promise_in_bounds")
            mend = (lanes == 15) | (dk != dnext)
            dprev = dk.at[jnp.maximum(lanes - 1, 0)].get(
                mode="promise_in_bounds")
            sbeg = (lanes == 0) | (dk != dprev)
            sidx = plsc.cummax(jnp.where(sbeg, lanes, 0))
            pprev = psum.at[jnp.maximum(sidx - 1, 0)].get(
                mode="promise_in_bounds")
            runsum = psum - jnp.where(sidx == 0, 0.0, pprev)
            plsc.addupdate_scatter(den_v, [dk], runsum, mask=mend)

        def dmove(v, carry3):
            dstSs[q][pl.ds(v * 16, 16)] = dsts[q][pl.ds(v * 16, 16)]
            return carry3

        lax.fori_loop(0, CH // 16, dmove, 0)

        @pl.when(b >= 1)
        def _():
            scatter_wait((q + 3) % NPAR)

        scatter_issue(q)

        @pl.when(b < NB - 4)
        def _():
            idx_issue(b + 4, q)

    def block_body(b, carry):
        for qq in range(NPAR):
            @pl.when(b % NPAR == qq)
            def _(qq=qq):
                block_steps(b, qq)

        return carry

    lax.fori_loop(0, NB, block_body, 0)
    scatter_wait((NB - 1) % NPAR)

    @pl.when(c == 0)
    def _():
        pltpu.sync_copy(den_v, dpart.at[s])

    plsc.subcore_barrier()

    @pl.when(c == 0)
    def _():
        pltpu.sync_copy(acc.at[pl.ds(s * RPS, RPS)],
                        out0.at[pl.ds(s * RPS, RPS)])

    @pl.when(c == 1)
    def _():
        pltpu.sync_copy(acc.at[pl.ds(s * RPS, RPS)],
                        out1.at[pl.ds(s * RPS, RPS)])


def kernel(obs, edge_index, W1, as1, ad1, b1, W2, as2, ad2, b2, W3, b3, W4,
           b4):
    loop = jnp.arange(N, dtype=jnp.int32)
    pad = jnp.zeros((EP - ET,), jnp.int32)
    src = jnp.concatenate([edge_index[0].astype(jnp.int32), loop, pad])
    dst = jnp.concatenate([edge_index[1].astype(jnp.int32), loop, pad])
    z = jnp.zeros((NR, HALF), F32)
    obs_p = jnp.pad(obs, ((0, NR - N), (0, 0)))

    as1r = as1.reshape(1, DH)
    ad1r = ad1.reshape(1, DH)
    as2r = as2.reshape(1, DH)
    ad2r = ad2.reshape(1, DH)
    b1r = b1.reshape(1, DH)
    b2r = b2.reshape(1, DH)
    b3r = b3.reshape(1, DH)
    w4p = jnp.pad(W4, ((0, 0), (0, HALF - DA)))
    b4p = jnp.pad(b4, (0, HALF - DA)).reshape(1, HALF)

    hA, hB, al = _stage1(obs_p, W1, as1r, ad1r)
    o0, o1, d1 = _sc_aggregate(hA, hB, al.reshape(2 * NR), src, dst, z)
    hA2, hB2, al2 = _stage2(o0, o1, d1, b1r, W2, as2r, ad2r)
    p0, p1, d2 = _sc_aggregate(hA2, hB2, al2.reshape(2 * NR), src, dst, z)
    act = _stage3(p0, p1, d2, b2r, W3, b3r, w4p, b4p)
    return act[:N, :DA]
